# Initial kernel scaffold; baseline (speedup 1.0000x reference)
import jax
import jax.numpy as jnp
from jax.experimental import pallas as pl


def kernel(x, batch):
    raise NotImplementedError("write your pallas kernel here")


# speedup vs baseline: 3.9083x; 3.9083x over previous
import functools

import jax
import jax.numpy as jnp
from jax import lax
from jax.experimental import pallas as pl
from jax.experimental.pallas import tpu as pltpu
from jax.experimental.pallas import tpu_sc as plsc

N = 320000
F = 128
G = 256

_INFO = plsc.get_sparse_core_info()
NC = _INFO.num_cores
NS = _INFO.num_subcores
NW = NC * NS
ROWS_PER_W = N // NW
CHUNK = 80
NCHUNK = ROWS_PER_W // CHUNK
ROWS_PER_TILE_OUT = G // NS


def _sc_partial_kernel(x_hbm, b_hbm, out_hbm, xbuf, ibuf, obuf, acc):
    c = lax.axis_index("c")
    s = lax.axis_index("s")
    wid = s * NC + c
    base = wid * ROWS_PER_W

    for r in range(ROWS_PER_TILE_OUT):
        for j in range(F // 16):
            obuf[r, pl.ds(j * 16, 16)] = jnp.zeros((16,), jnp.float32)
    pltpu.sync_copy(obuf, acc.at[pl.ds(s * ROWS_PER_TILE_OUT, ROWS_PER_TILE_OUT)])
    plsc.subcore_barrier()

    def body(i, carry):
        start = base + i * CHUNK
        pltpu.sync_copy(x_hbm.at[pl.ds(start, CHUNK)], xbuf)
        pltpu.sync_copy(b_hbm.at[pl.ds(start, CHUNK)], ibuf)
        pltpu.sync_copy(xbuf, acc.at[ibuf], add=True)
        return carry

    lax.fori_loop(0, NCHUNK, body, 0)
    plsc.subcore_barrier()

    row0 = s * ROWS_PER_TILE_OUT
    pltpu.sync_copy(acc.at[pl.ds(row0, ROWS_PER_TILE_OUT)], obuf)
    pltpu.sync_copy(obuf, out_hbm.at[c, pl.ds(row0, ROWS_PER_TILE_OUT)])


@jax.jit
def _sc_partials(x, batch):
    mesh = plsc.VectorSubcoreMesh(core_axis_name="c", subcore_axis_name="s")
    return pl.kernel(
        _sc_partial_kernel,
        mesh=mesh,
        out_type=jax.ShapeDtypeStruct((NC, G, F), jnp.float32),
        scratch_types=[
            pltpu.VMEM((CHUNK, F), jnp.float32),
            pltpu.VMEM((CHUNK,), jnp.int32),
            pltpu.VMEM((ROWS_PER_TILE_OUT, F), jnp.float32),
            pltpu.VMEM_SHARED((G, F), jnp.float32),
        ],
    )(x, batch)


def _combine_kernel(p_ref, o_ref):
    o_ref[...] = p_ref[0] + p_ref[1]


@jax.jit
def _combine(partials):
    return pl.pallas_call(
        _combine_kernel,
        out_shape=jax.ShapeDtypeStruct((G, F), jnp.float32),
    )(partials)


def kernel(x, batch):
    partials = _sc_partials(x, batch.astype(jnp.int32))
    return _combine(partials)


# speedup vs baseline: 5.8117x; 1.4870x over previous
import jax
import jax.numpy as jnp
from jax import lax
from jax.experimental import pallas as pl
from jax.experimental.pallas import tpu as pltpu
from jax.experimental.pallas import tpu_sc as plsc

N = 320000
F = 128
G = 256

_INFO = plsc.get_sparse_core_info()
NC = _INFO.num_cores
NS = _INFO.num_subcores
NW = NC * NS
ROWS_PER_W = N // NW
CHUNK = 80
NCHUNK = ROWS_PER_W // CHUNK
ROWS_PER_TILE_OUT = G // NS


def _sc_partial_kernel(x_hbm, b_hbm, out_hbm, xbufa, xbufb, ibuf, obuf, acc,
                       sema, semb):
    c = lax.axis_index("c")
    s = lax.axis_index("s")
    wid = s * NC + c
    base = wid * ROWS_PER_W

    for r in range(ROWS_PER_TILE_OUT):
        for j in range(F // 16):
            obuf[r, pl.ds(j * 16, 16)] = jnp.zeros((16,), jnp.float32)
    pltpu.sync_copy(obuf, acc.at[pl.ds(s * ROWS_PER_TILE_OUT, ROWS_PER_TILE_OUT)])
    pltpu.sync_copy(b_hbm.at[wid], ibuf)
    plsc.subcore_barrier()

    def gather_start(chunk_idx, buf, sem):
        return pltpu.async_copy(
            x_hbm.at[pl.ds(base + chunk_idx * CHUNK, CHUNK)], buf, sem)

    def gather_wait(chunk_idx, buf, sem):
        pltpu.make_async_copy(
            x_hbm.at[pl.ds(base + chunk_idx * CHUNK, CHUNK)], buf, sem).wait()

    def scatter_add(chunk_idx, buf):
        pltpu.sync_copy(buf, acc.at[ibuf.at[chunk_idx]], add=True)

    gather_start(0, xbufa, sema)

    def body(i, carry):
        c0 = 2 * i
        gather_wait(c0, xbufa, sema)
        gather_start(c0 + 1, xbufb, semb)
        scatter_add(c0, xbufa)
        gather_wait(c0 + 1, xbufb, semb)
        gather_start(c0 + 2, xbufa, sema)
        scatter_add(c0 + 1, xbufb)
        return carry

    lax.fori_loop(0, (NCHUNK - 1) // 2, body, 0)
    gather_wait(NCHUNK - 1, xbufa, sema)
    scatter_add(NCHUNK - 1, xbufa)
    plsc.subcore_barrier()

    row0 = s * ROWS_PER_TILE_OUT
    pltpu.sync_copy(acc.at[pl.ds(row0, ROWS_PER_TILE_OUT)], obuf)
    pltpu.sync_copy(obuf, out_hbm.at[c, pl.ds(row0, ROWS_PER_TILE_OUT)])


@jax.jit
def _sc_partials(x, batch_blocked):
    mesh = plsc.VectorSubcoreMesh(core_axis_name="c", subcore_axis_name="s")
    return pl.kernel(
        _sc_partial_kernel,
        mesh=mesh,
        out_type=jax.ShapeDtypeStruct((NC, G, F), jnp.float32),
        scratch_types=[
            pltpu.VMEM((CHUNK, F), jnp.float32),
            pltpu.VMEM((CHUNK, F), jnp.float32),
            pltpu.VMEM((NCHUNK, CHUNK), jnp.int32),
            pltpu.VMEM((ROWS_PER_TILE_OUT, F), jnp.float32),
            pltpu.VMEM_SHARED((G, F), jnp.float32),
            pltpu.SemaphoreType.DMA,
            pltpu.SemaphoreType.DMA,
        ],
    )(x, batch_blocked)


def _combine_kernel(p_ref, o_ref):
    o_ref[...] = p_ref[0] + p_ref[1]


@jax.jit
def _combine(partials):
    return pl.pallas_call(
        _combine_kernel,
        out_shape=jax.ShapeDtypeStruct((G, F), jnp.float32),
    )(partials)


def kernel(x, batch):
    batch_blocked = batch.astype(jnp.int32).reshape(NW, NCHUNK, CHUNK)
    partials = _sc_partials(x, batch_blocked)
    return _combine(partials)


# speedup vs baseline: 6.1428x; 1.0570x over previous
import jax
import jax.numpy as jnp
from jax import lax
from jax.experimental import pallas as pl
from jax.experimental.pallas import tpu as pltpu
from jax.experimental.pallas import tpu_sc as plsc

N = 320000
F = 128
G = 256

_INFO = plsc.get_sparse_core_info()
NC = _INFO.num_cores
NS = _INFO.num_subcores
NW = NC * NS
CHUNK = 128
TOTCHUNK = N // CHUNK
NCHUNK = TOTCHUNK // NW
EXTRA = TOTCHUNK - NCHUNK * NW
ROWS_PER_W = NCHUNK * CHUNK
ROWS_PER_TILE_OUT = G // NS


def _sc_partial_kernel(x_hbm, b_hbm, out_hbm, xbufa, xbufb, ibuf, obuf, acc,
                       sema, semb):
    c = lax.axis_index("c")
    s = lax.axis_index("s")
    wid = s * NC + c
    base = wid * ROWS_PER_W

    for r in range(ROWS_PER_TILE_OUT):
        for j in range(F // 16):
            obuf[r, pl.ds(j * 16, 16)] = jnp.zeros((16,), jnp.float32)
    pltpu.sync_copy(obuf, acc.at[pl.ds(s * ROWS_PER_TILE_OUT, ROWS_PER_TILE_OUT)])
    pltpu.sync_copy(b_hbm.at[wid], ibuf)
    plsc.subcore_barrier()

    def gather_start(chunk_idx, buf, sem):
        return pltpu.async_copy(
            x_hbm.at[pl.ds(base + chunk_idx * CHUNK, CHUNK)], buf, sem)

    def gather_wait(chunk_idx, buf, sem):
        pltpu.make_async_copy(
            x_hbm.at[pl.ds(base + chunk_idx * CHUNK, CHUNK)], buf, sem).wait()

    def scatter_add(chunk_idx, buf):
        pltpu.sync_copy(buf, acc.at[ibuf.at[chunk_idx]], add=True)

    gather_start(0, xbufa, sema)

    def body(i, carry):
        c0 = 2 * i
        gather_wait(c0, xbufa, sema)
        gather_start(c0 + 1, xbufb, semb)
        scatter_add(c0, xbufa)
        gather_wait(c0 + 1, xbufb, semb)
        gather_start(c0 + 2, xbufa, sema)
        scatter_add(c0 + 1, xbufb)
        return carry

    lax.fori_loop(0, NCHUNK // 2, body, 0)
    gather_wait(NCHUNK, xbufa, sema)

    @pl.when(wid < EXTRA)
    def _():
        start = NW * ROWS_PER_W + wid * CHUNK
        pltpu.sync_copy(x_hbm.at[pl.ds(start, CHUNK)], xbufb)
        pltpu.sync_copy(xbufb, acc.at[ibuf.at[NCHUNK]], add=True)

    plsc.subcore_barrier()

    row0 = s * ROWS_PER_TILE_OUT
    pltpu.sync_copy(acc.at[pl.ds(row0, ROWS_PER_TILE_OUT)], obuf)
    pltpu.sync_copy(obuf, out_hbm.at[c, pl.ds(row0, ROWS_PER_TILE_OUT)])


@jax.jit
def _sc_partials(x, batch_blocked):
    mesh = plsc.VectorSubcoreMesh(core_axis_name="c", subcore_axis_name="s")
    return pl.kernel(
        _sc_partial_kernel,
        mesh=mesh,
        out_type=jax.ShapeDtypeStruct((NC, G, F), jnp.float32),
        scratch_types=[
            pltpu.VMEM((CHUNK, F), jnp.float32),
            pltpu.VMEM((CHUNK, F), jnp.float32),
            pltpu.VMEM((NCHUNK + 1, CHUNK), jnp.int32),
            pltpu.VMEM((ROWS_PER_TILE_OUT, F), jnp.float32),
            pltpu.VMEM_SHARED((G, F), jnp.float32),
            pltpu.SemaphoreType.DMA,
            pltpu.SemaphoreType.DMA,
        ],
    )(x, batch_blocked)


def _combine_kernel(p_ref, o_ref):
    o_ref[...] = p_ref[0] + p_ref[1]


@jax.jit
def _combine(partials):
    return pl.pallas_call(
        _combine_kernel,
        out_shape=jax.ShapeDtypeStruct((G, F), jnp.float32),
    )(partials)


def kernel(x, batch):
    b = batch.astype(jnp.int32).reshape(TOTCHUNK, CHUNK)
    slabs = b[:NW * NCHUNK].reshape(NW, NCHUNK, CHUNK)
    extras = jnp.concatenate(
        [b[NW * NCHUNK:], jnp.zeros((NW - EXTRA, CHUNK), jnp.int32)]
    ).reshape(NW, 1, CHUNK)
    batch_blocked = jnp.concatenate([slabs, extras], axis=1)
    partials = _sc_partials(x, batch_blocked)
    return _combine(partials)
